# baseline (speedup 1.0000x reference)
import jax
import jax.numpy as jnp
from jax import lax
from jax.experimental import pallas as pl

S = 32
D = 128
DT = 64
NSTEP = 5
TEMP = 0.1
NSINK = 20
P = 8
H = 2
NPB = 2 * S * P
EPB = 256 * P

_f32 = jnp.float32
_bf16 = jnp.bfloat16


def _bdot(a, b):
    return lax.dot_general(a.astype(_bf16), b.astype(_bf16),
                           (((1,), (0,)), ((), ())),
                           preferred_element_type=_f32)


def _bdot3(a, b, dims):
    return lax.dot_general(a.astype(_bf16), b.astype(_bf16), dims,
                           preferred_element_type=_f32)


def _init(nf, ef, locf_c, locf_r, loct_c, loct_r, W_ne, b_ne, W_ee, b_ee):
    x = _bdot(nf, W_ne) + b_ne
    efp = _bdot(ef, W_ee) + b_ee
    iota = lax.broadcasted_iota(jnp.int32, (EPB, NPB), 1)
    oh_f = (iota == locf_c).astype(_bf16)
    oh_t = (iota == loct_c).astype(_bf16)
    iota_t = lax.broadcasted_iota(jnp.int32, (NPB, EPB), 0)
    oh_fT = (iota_t == locf_r).astype(_bf16)
    oh_tT = (iota_t == loct_r).astype(_bf16)
    return dict(x=x, store=jnp.zeros_like(x), efp=efp, oh_f=oh_f, oh_t=oh_t,
                oh_fT=oh_fT, oh_tT=oh_tT, q3=None, qf3=None)


def _step(st, Wc1, bc1, Wc2, bc2, Wm1, bm1, Wm2, bm2,
          Wu1, bu1, Wu2, bu2, Wt1, bt1, Wt2, bt2):
    relu = lambda v: jnp.maximum(v, 0.0)
    nn = (((1,), (0,)), ((), ()))
    efp = st['efp']

    comb = _bdot(relu(_bdot(jnp.concatenate([st['x'], st['store']], axis=1),
                            Wc1) + bc1), Wc2) + bc2
    cbf = comb.astype(_bf16)
    fsb = lax.dot_general(st['oh_f'], cbf, nn, preferred_element_type=_f32)
    tsb = lax.dot_general(st['oh_t'], cbf, nn, preferred_element_type=_f32)
    msg = _bdot(relu(_bdot(jnp.concatenate([fsb, tsb, efp], axis=1),
                           Wm1) + bm1), Wm2) + bm2
    rmsg = _bdot(relu(_bdot(jnp.concatenate([tsb, fsb, efp], axis=1),
                            Wm1) + bm1), Wm2) + bm2

    def scat(ohT, m):
        m0 = m.astype(_bf16)
        r1 = m - m0.astype(_f32)
        m1 = r1.astype(_bf16)
        m2 = (r1 - m1.astype(_f32)).astype(_bf16)
        acc = lax.dot_general(ohT, m0, nn, preferred_element_type=_f32)
        acc = acc + lax.dot_general(ohT, m1, nn, preferred_element_type=_f32)
        acc = acc + lax.dot_general(ohT, m2, nn, preferred_element_type=_f32)
        return acc

    agg = scat(st['oh_tT'], msg)
    ragg = scat(st['oh_fT'], rmsg)
    x = _bdot(relu(_bdot(jnp.concatenate([comb, agg, ragg], axis=1),
                         Wu1) + bu1), Wu2) + bu2

    x4 = x.reshape(P, 2 * S, D)
    q3 = x4[:, :S, :]
    c3 = x4[:, S:, :]
    t_all = _bdot(relu(_bdot(x, Wt1) + bt1), Wt2) + bt2
    t4 = t_all.reshape(P, 2 * S, DT)
    tq3 = t4[:, :S, :]
    tc3 = t4[:, S:, :]

    la = _bdot3(tq3, tc3, (((2,), (2,)), ((0,), (0,)))) * (1.0 / TEMP)
    for _ in range(NSINK):
        la2 = la.reshape(P * S, S)
        m1 = jnp.max(la2, axis=1, keepdims=True)
        la2 = la2 - (jnp.log(jnp.sum(jnp.exp(la2 - m1), axis=1,
                                     keepdims=True)) + m1)
        la = la2.reshape(P, S, S)
        m0 = jnp.max(la, axis=1, keepdims=True)
        la = la - (jnp.log(jnp.sum(jnp.exp(la - m0), axis=1,
                                   keepdims=True)) + m0)
    plan = jnp.exp(la)
    qf3 = _bdot3(plan, c3, (((2,), (1,)), ((0,), (0,))))
    cf3 = _bdot3(plan, q3, (((1,), (1,)), ((0,), (0,))))
    st = dict(st)
    st['x'] = x
    st['store'] = jnp.concatenate([qf3, cf3], axis=1).reshape(NPB, D)
    st['q3'] = q3
    st['qf3'] = qf3
    return st


def _body(nf, ef, locf, loct, locfr, loctr,
          W_ne, b_ne, W_ee, b_ee, Wc1, bc1, Wc2, bc2,
          Wm1, bm1, Wm2, bm2, Wu1, bu1, Wu2, bu2,
          Wt1, bt1, Wt2, bt2, out):
    sw = (Wc1[...], bc1[...], Wc2[...], bc2[...], Wm1[...], bm1[...],
          Wm2[...], bm2[...], Wu1[...], bu1[...], Wu2[...], bu2[...],
          Wt1[...], bt1[...], Wt2[...], bt2[...])
    sts = [_init(nf[h * NPB:(h + 1) * NPB],
                 ef[h * EPB:(h + 1) * EPB],
                 locf[0][h * EPB:(h + 1) * EPB],
                 locfr[0][:, h * EPB:(h + 1) * EPB],
                 loct[0][h * EPB:(h + 1) * EPB],
                 loctr[0][:, h * EPB:(h + 1) * EPB],
                 W_ne[...], b_ne[...], W_ee[...], b_ee[...])
           for h in range(H)]
    for _ in range(NSTEP):
        for h in range(H):
            sts[h] = _step(sts[h], *sw)
    ss = [-jnp.sum(jnp.maximum(st['q3'] - st['qf3'], 0.0), axis=(1, 2))
          for st in sts]
    out[...] = jnp.concatenate(ss).reshape(1, 1, H * P)


def kernel(node_features, edge_features, from_idx, to_idx,
           W_ne, b_ne, W_ee, b_ee, Wc1, bc1, Wc2, bc2,
           Wm1, bm1, Wm2, bm2, Wu1, bu1, Wu2, bu2,
           Wt1, bt1, Wt2, bt2):
    N = node_features.shape[0]
    NB = N // (H * NPB)
    locf = (from_idx % NPB).astype(jnp.int32).reshape(NB, H * EPB, 1)
    loct = (to_idx % NPB).astype(jnp.int32).reshape(NB, H * EPB, 1)
    locfr = (from_idx % NPB).astype(jnp.int32).reshape(NB, 1, H * EPB)
    loctr = (to_idx % NPB).astype(jnp.int32).reshape(NB, 1, H * EPB)

    row = lambda v: v.reshape(1, -1)
    ops = [
        (node_features, pl.BlockSpec((H * NPB, node_features.shape[1]),
                                     lambda i: (i, 0))),
        (edge_features, pl.BlockSpec((H * EPB, edge_features.shape[1]),
                                     lambda i: (i, 0))),
        (locf, pl.BlockSpec((1, H * EPB, 1), lambda i: (i, 0, 0))),
        (loct, pl.BlockSpec((1, H * EPB, 1), lambda i: (i, 0, 0))),
        (locfr, pl.BlockSpec((1, 1, H * EPB), lambda i: (i, 0, 0))),
        (loctr, pl.BlockSpec((1, 1, H * EPB), lambda i: (i, 0, 0))),
    ]
    for w in (W_ne, row(b_ne), W_ee, row(b_ee), Wc1, row(bc1), Wc2, row(bc2),
              Wm1, row(bm1), Wm2, row(bm2), Wu1, row(bu1), Wu2, row(bu2),
              Wt1, row(bt1), Wt2, row(bt2)):
        ops.append((w, pl.BlockSpec(w.shape, lambda i: (0, 0))))

    out = pl.pallas_call(
        _body,
        grid=(NB,),
        in_specs=[spec for _, spec in ops],
        out_specs=pl.BlockSpec((1, 1, H * P), lambda i: (i, 0, 0)),
        out_shape=jax.ShapeDtypeStruct((NB, 1, H * P), _f32),
    )(*[a for a, _ in ops])
    return out.reshape(NB * H * P)


# speedup vs baseline: 1.0017x; 1.0017x over previous
import jax
import jax.numpy as jnp
from jax import lax
from jax.experimental import pallas as pl

S = 32
D = 128
DT = 64
NSTEP = 5
TEMP = 0.1
NSINK = 20
P = 8
H = 2
NPB = 2 * S * P
EPB = 256 * P

_f32 = jnp.float32
_bf16 = jnp.bfloat16


def _bdot(a, b):
    return lax.dot_general(a.astype(_bf16), b.astype(_bf16),
                           (((1,), (0,)), ((), ())),
                           preferred_element_type=_f32)


def _bdot3(a, b, dims):
    return lax.dot_general(a.astype(_bf16), b.astype(_bf16), dims,
                           preferred_element_type=_f32)


def _init(nf, ef, locf_c, locf_r, loct_c, loct_r, W_ne, b_ne, W_ee, b_ee):
    x = _bdot(nf, W_ne) + b_ne
    efp = _bdot(ef, W_ee) + b_ee
    iota = lax.broadcasted_iota(jnp.int32, (EPB, NPB), 1)
    oh_f = (iota == locf_c).astype(_bf16)
    oh_t = (iota == loct_c).astype(_bf16)
    iota_t = lax.broadcasted_iota(jnp.int32, (NPB, EPB), 0)
    oh_fT = (iota_t == locf_r).astype(_bf16)
    oh_tT = (iota_t == loct_r).astype(_bf16)
    return dict(x=x, store=jnp.zeros_like(x), efp=efp, oh_f=oh_f, oh_t=oh_t,
                oh_fT=oh_fT, oh_tT=oh_tT, q3=None, qf3=None)


def _step(st, Wc1, bc1, Wc2, bc2, Wm1, bm1, Wm2, bm2,
          Wu1, bu1, Wu2, bu2, Wt1, bt1, Wt2, bt2):
    relu = lambda v: jnp.maximum(v, 0.0)
    nn = (((1,), (0,)), ((), ()))
    efp = st['efp']

    comb = _bdot(relu(_bdot(jnp.concatenate([st['x'], st['store']], axis=1),
                            Wc1) + bc1), Wc2) + bc2
    cbf = comb.astype(_bf16)
    fsb = lax.dot_general(st['oh_f'], cbf, nn, preferred_element_type=_f32)
    tsb = lax.dot_general(st['oh_t'], cbf, nn, preferred_element_type=_f32)
    msg = _bdot(relu(_bdot(jnp.concatenate([fsb, tsb, efp], axis=1),
                           Wm1) + bm1), Wm2) + bm2
    rmsg = _bdot(relu(_bdot(jnp.concatenate([tsb, fsb, efp], axis=1),
                            Wm1) + bm1), Wm2) + bm2

    def scat(ohT, m):
        m0 = m.astype(_bf16)
        r1 = m - m0.astype(_f32)
        m1 = r1.astype(_bf16)
        m2 = (r1 - m1.astype(_f32)).astype(_bf16)
        acc = lax.dot_general(ohT, m0, nn, preferred_element_type=_f32)
        acc = acc + lax.dot_general(ohT, m1, nn, preferred_element_type=_f32)
        acc = acc + lax.dot_general(ohT, m2, nn, preferred_element_type=_f32)
        return acc

    agg = scat(st['oh_tT'], msg)
    ragg = scat(st['oh_fT'], rmsg)
    x = _bdot(relu(_bdot(jnp.concatenate([comb, agg, ragg], axis=1),
                         Wu1) + bu1), Wu2) + bu2

    x4 = x.reshape(P, 2 * S, D)
    q3 = x4[:, :S, :]
    c3 = x4[:, S:, :]
    t_all = _bdot(relu(_bdot(x, Wt1) + bt1), Wt2) + bt2
    t4 = t_all.reshape(P, 2 * S, DT)
    tq3 = t4[:, :S, :]
    tc3 = t4[:, S:, :]

    la = _bdot3(tq3, tc3, (((2,), (2,)), ((0,), (0,)))) * (1.0 / TEMP)
    for _ in range(NSINK):
        m1 = jnp.max(la, axis=2, keepdims=True)
        la = la - (jnp.log(jnp.sum(jnp.exp(la - m1), axis=2,
                                   keepdims=True)) + m1)
        m0 = jnp.max(la, axis=1, keepdims=True)
        la = la - (jnp.log(jnp.sum(jnp.exp(la - m0), axis=1,
                                   keepdims=True)) + m0)
    plan = jnp.exp(la)
    qf3 = _bdot3(plan, c3, (((2,), (1,)), ((0,), (0,))))
    cf3 = _bdot3(plan, q3, (((1,), (1,)), ((0,), (0,))))
    st = dict(st)
    st['x'] = x
    st['store'] = jnp.concatenate([qf3, cf3], axis=1).reshape(NPB, D)
    st['q3'] = q3
    st['qf3'] = qf3
    return st


def _body(nf, ef, locf, loct, locfr, loctr,
          W_ne, b_ne, W_ee, b_ee, Wc1, bc1, Wc2, bc2,
          Wm1, bm1, Wm2, bm2, Wu1, bu1, Wu2, bu2,
          Wt1, bt1, Wt2, bt2, out):
    sw = (Wc1[...], bc1[...], Wc2[...], bc2[...], Wm1[...], bm1[...],
          Wm2[...], bm2[...], Wu1[...], bu1[...], Wu2[...], bu2[...],
          Wt1[...], bt1[...], Wt2[...], bt2[...])
    sts = [_init(nf[h * NPB:(h + 1) * NPB],
                 ef[h * EPB:(h + 1) * EPB],
                 locf[0][h * EPB:(h + 1) * EPB],
                 locfr[0][:, h * EPB:(h + 1) * EPB],
                 loct[0][h * EPB:(h + 1) * EPB],
                 loctr[0][:, h * EPB:(h + 1) * EPB],
                 W_ne[...], b_ne[...], W_ee[...], b_ee[...])
           for h in range(H)]
    for _ in range(NSTEP):
        for h in range(H):
            sts[h] = _step(sts[h], *sw)
    ss = [-jnp.sum(jnp.maximum(st['q3'] - st['qf3'], 0.0), axis=(1, 2))
          for st in sts]
    out[...] = jnp.concatenate(ss).reshape(1, 1, H * P)


def kernel(node_features, edge_features, from_idx, to_idx,
           W_ne, b_ne, W_ee, b_ee, Wc1, bc1, Wc2, bc2,
           Wm1, bm1, Wm2, bm2, Wu1, bu1, Wu2, bu2,
           Wt1, bt1, Wt2, bt2):
    N = node_features.shape[0]
    NB = N // (H * NPB)
    locf = (from_idx % NPB).astype(jnp.int32).reshape(NB, H * EPB, 1)
    loct = (to_idx % NPB).astype(jnp.int32).reshape(NB, H * EPB, 1)
    locfr = (from_idx % NPB).astype(jnp.int32).reshape(NB, 1, H * EPB)
    loctr = (to_idx % NPB).astype(jnp.int32).reshape(NB, 1, H * EPB)

    row = lambda v: v.reshape(1, -1)
    ops = [
        (node_features, pl.BlockSpec((H * NPB, node_features.shape[1]),
                                     lambda i: (i, 0))),
        (edge_features, pl.BlockSpec((H * EPB, edge_features.shape[1]),
                                     lambda i: (i, 0))),
        (locf, pl.BlockSpec((1, H * EPB, 1), lambda i: (i, 0, 0))),
        (loct, pl.BlockSpec((1, H * EPB, 1), lambda i: (i, 0, 0))),
        (locfr, pl.BlockSpec((1, 1, H * EPB), lambda i: (i, 0, 0))),
        (loctr, pl.BlockSpec((1, 1, H * EPB), lambda i: (i, 0, 0))),
    ]
    for w in (W_ne, row(b_ne), W_ee, row(b_ee), Wc1, row(bc1), Wc2, row(bc2),
              Wm1, row(bm1), Wm2, row(bm2), Wu1, row(bu1), Wu2, row(bu2),
              Wt1, row(bt1), Wt2, row(bt2)):
        ops.append((w, pl.BlockSpec(w.shape, lambda i: (0, 0))))

    out = pl.pallas_call(
        _body,
        grid=(NB,),
        in_specs=[spec for _, spec in ops],
        out_specs=pl.BlockSpec((1, 1, H * P), lambda i: (i, 0, 0)),
        out_shape=jax.ShapeDtypeStruct((NB, 1, H * P), _f32),
    )(*[a for a, _ in ops])
    return out.reshape(NB * H * P)
